# Initial kernel scaffold; baseline (speedup 1.0000x reference)
import jax
import jax.numpy as jnp
from jax.experimental import pallas as pl


def kernel(x_seq, Wl1, bl1, Wr1, br1, att1, bias1, Wl2, bl2, Wr2, br2, att2, bias2, Wout, bout):
    raise NotImplementedError("write your pallas kernel here")


# speedup vs baseline: 557.9969x; 557.9969x over previous
import functools

import jax
import jax.numpy as jnp
import numpy as np
from jax.experimental import pallas as pl

J = 17
H = 8
C = 16
F = H * C

_EXP = np.kron(np.eye(H, dtype=np.float32), np.ones((1, C), dtype=np.float32))


def _elu(z):
    return jnp.where(z > 0, z, jnp.exp(jnp.minimum(z, 0.0)) - 1.0)


def _gat_block(x2, Wl, bl, Wr, br, A, bias, EXP, Gb):
    xl = jnp.dot(x2, Wl, preferred_element_type=jnp.float32) + bl
    xr = jnp.dot(x2, Wr, preferred_element_type=jnp.float32) + br
    xl3 = xl.reshape(J, Gb, F)
    xr3 = xr.reshape(J, Gb, F)
    u = xl3[None, :, :, :] + xr3[:, None, :, :]
    e = jnp.maximum(u, 0.2 * u)
    logits = jnp.dot(e.reshape(J * J * Gb, F), A,
                     preferred_element_type=jnp.float32)
    l4 = logits.reshape(J, J, Gb, H)
    m = jnp.max(l4, axis=1, keepdims=True)
    p = jnp.exp(l4 - m)
    s = jnp.sum(p, axis=1, keepdims=True)
    alpha = p / (s + 1e-16)
    a128 = jnp.dot(alpha.reshape(J * J * Gb, H), EXP,
                   preferred_element_type=jnp.float32).reshape(J, J, Gb, F)
    h = jnp.sum(a128 * xl3[None, :, :, :], axis=1)
    return h + bias.reshape(1, 1, F)


def _body(x_ref, Wl1_ref, bl1_ref, Wr1_ref, br1_ref, A1_ref, bias1_ref,
          Wl2_ref, bl2_ref, Wr2_ref, br2_ref, A2_ref, bias2_ref,
          EXP_ref, WoutR_ref, bout_ref, out_ref, *, Gb):
    x = x_ref[...]
    EXP = EXP_ref[...]
    h1 = _gat_block(x.reshape(J * Gb, 3),
                    Wl1_ref[...], bl1_ref[...], Wr1_ref[...], br1_ref[...],
                    A1_ref[...], bias1_ref[...], EXP, Gb)
    h1 = _elu(h1)
    h2 = _gat_block(h1.reshape(J * Gb, F),
                    Wl2_ref[...], bl2_ref[...], Wr2_ref[...], br2_ref[...],
                    A2_ref[...], bias2_ref[...], EXP, Gb)
    h2 = _elu(h2)
    acc = jnp.dot(h2[0], WoutR_ref[0], preferred_element_type=jnp.float32)
    for j in range(1, J):
        acc = acc + jnp.dot(h2[j], WoutR_ref[j],
                            preferred_element_type=jnp.float32)
    out_ref[...] = acc + bout_ref[...]


def kernel(x_seq, Wl1, bl1, Wr1, br1, att1, bias1,
           Wl2, bl2, Wr2, br2, att2, bias2, Wout, bout):
    B, T, _ = x_seq.shape
    G = B * T
    Gb = 64
    assert G % Gb == 0

    x3 = x_seq.reshape(G, J, 3).transpose(1, 0, 2)

    EXP = jnp.asarray(_EXP)
    A1 = att1.reshape(F, 1) * EXP.T
    A2 = att2.reshape(F, 1) * EXP.T
    WoutR = Wout.reshape(J, F, 256)

    full = lambda shape: pl.BlockSpec(shape, lambda g, s=None: (0,) * len(shape))
    out = pl.pallas_call(
        functools.partial(_body, Gb=Gb),
        grid=(G // Gb,),
        in_specs=[
            pl.BlockSpec((J, Gb, 3), lambda g: (0, g, 0)),
            full((3, F)), full((1, F)), full((3, F)), full((1, F)),
            full((F, H)), full((1, F)),
            full((F, F)), full((1, F)), full((F, F)), full((1, F)),
            full((F, H)), full((1, F)),
            full((H, F)), full((J, F, 256)), full((1, 256)),
        ],
        out_specs=pl.BlockSpec((Gb, 256), lambda g: (g, 0)),
        out_shape=jax.ShapeDtypeStruct((G, 256), jnp.float32),
    )(x3, Wl1, bl1.reshape(1, F), Wr1, br1.reshape(1, F), A1,
      bias1.reshape(1, F),
      Wl2, bl2.reshape(1, F), Wr2, br2.reshape(1, F), A2,
      bias2.reshape(1, F),
      EXP, WoutR, bout.reshape(1, 256))
    return out.reshape(B, T, 256)
